# baseline (speedup 1.0000x reference)
import functools

import numpy as np
import jax
import jax.numpy as jnp
from jax import lax
from jax.experimental import pallas as pl
from jax.experimental.pallas import tpu as pltpu, tpu_sc as plsc

_NUM_SEGMENTS = 4096


@functools.lru_cache(maxsize=None)
def _segment_reps(n: int):
    if n <= _NUM_SEGMENTS:
        return np.linspace(0, n - 1, _NUM_SEGMENTS, dtype=int).astype(np.int32)
    idx = np.linspace(0, n - 1, n, dtype=int)
    chunks = np.array_split(idx, _NUM_SEGMENTS)
    rng = np.random.default_rng(0)
    return np.array([rng.choice(c, 1) for c in chunks]).squeeze().astype(np.int32)


@functools.lru_cache(maxsize=None)
def _make_sc_gather(V: int, D: int, B: int):
    info = plsc.get_sparse_core_info()
    nw = info.num_cores * info.num_subcores
    assert B % nw == 0
    b_per_w = B // nw
    mesh = plsc.VectorSubcoreMesh(core_axis_name="c", subcore_axis_name="s")

    @functools.partial(
        pl.kernel,
        mesh=mesh,
        out_type=jax.ShapeDtypeStruct((B, D), jnp.float32),
        scratch_types=[
            pltpu.VMEM((b_per_w,), jnp.int32),
            pltpu.VMEM((b_per_w, D), jnp.float32),
            pltpu.SemaphoreType.DMA,
        ],
        compiler_params=pltpu.CompilerParams(use_tc_tiling_on_sc=True),
    )
    def gather_kernel(x_hbm, idx_hbm, out_hbm, idx_v, out_v, sem):
        wid = lax.axis_index("s") * info.num_cores + lax.axis_index("c")
        base = wid * b_per_w
        pltpu.sync_copy(idx_hbm.at[pl.ds(base, b_per_w)], idx_v)

        def issue(g, carry):
            vec = idx_v[pl.ds(g * 16, 16)]
            for j in range(16):
                row = vec[j]
                pltpu.async_copy(x_hbm.at[row], out_v.at[g * 16 + j], sem)
            return carry

        lax.fori_loop(0, b_per_w // 16, issue, 0)
        pltpu.make_async_copy(x_hbm.at[pl.ds(0, b_per_w)], out_v, sem).wait()
        pltpu.sync_copy(out_v, out_hbm.at[pl.ds(base, b_per_w)])

    return gather_kernel


def kernel(x):
    n, d = x.shape
    ch = jnp.asarray(_segment_reps(n))
    return _make_sc_gather(n, d, _NUM_SEGMENTS)(x, ch)


# speedup vs baseline: 1.6947x; 1.6947x over previous
import functools

import numpy as np
import jax
import jax.numpy as jnp
from jax import lax
from jax.experimental import pallas as pl
from jax.experimental.pallas import tpu as pltpu, tpu_sc as plsc

_NUM_SEGMENTS = 4096


@functools.lru_cache(maxsize=None)
def _segment_reps(n: int):
    if n <= _NUM_SEGMENTS:
        return np.linspace(0, n - 1, _NUM_SEGMENTS, dtype=int).astype(np.int32)
    idx = np.linspace(0, n - 1, n, dtype=int)
    chunks = np.array_split(idx, _NUM_SEGMENTS)
    rng = np.random.default_rng(0)
    return np.array([rng.choice(c, 1) for c in chunks]).squeeze().astype(np.int32)


@functools.lru_cache(maxsize=None)
def _make_sc_gather(D: int, V: int, B: int, W: int):
    info = plsc.get_sparse_core_info()
    nw = info.num_cores * info.num_subcores
    b_per_w = B // nw
    jobs = 2
    b_per_j = b_per_w // jobs
    lo_max = ((V + 127) & ~127) - W
    mesh = plsc.VectorSubcoreMesh(core_axis_name="c", subcore_axis_name="s")

    @functools.partial(
        pl.kernel,
        mesh=mesh,
        out_type=jax.ShapeDtypeStruct((D, B), jnp.float32),
        scratch_types=[
            pltpu.VMEM((b_per_w,), jnp.int32),
            pltpu.VMEM((D, W), jnp.float32),
            pltpu.VMEM((D, b_per_w), jnp.float32),
            pltpu.SemaphoreType.DMA,
        ],
        compiler_params=pltpu.CompilerParams(needs_layout_passes=False),
    )
    def gather_kernel(xt_hbm, idx_hbm, out_hbm, idx_v, slab_v, out_v, sem):
        wid = lax.axis_index("s") * info.num_cores + lax.axis_index("c")
        base = wid * b_per_w
        pltpu.sync_copy(idx_hbm.at[pl.ds(base, b_per_w)], idx_v)

        def job(jj, carry):
            head = idx_v[pl.ds(jj * b_per_j, 16)]
            lo = pl.multiple_of(lax.min(head[0] & ~127, lo_max), 128)
            pltpu.async_copy(
                xt_hbm.at[:, pl.ds(lo, W)], slab_v, sem
            ).wait()
            for blk in range(b_per_j // 16):
                off = idx_v[pl.ds(jj * b_per_j + blk * 16, 16)] - lo
                for j in range(D):
                    row = jnp.full((16,), j, jnp.int32)
                    val = plsc.load_gather(slab_v, [row, off])
                    out_v[j, pl.ds(jj * b_per_j + blk * 16, 16)] = val
            return carry

        lax.fori_loop(0, jobs, job, 0)
        pltpu.sync_copy(out_v, out_hbm.at[:, pl.ds(base, b_per_w)])

    return gather_kernel


def kernel(x):
    n, d = x.shape
    ch = jnp.asarray(_segment_reps(n))
    span = int(np.max(_segment_reps(n)[63:] - _segment_reps(n)[:-63])) + 1
    w = (span + 127 + 127) & ~127
    out_t = _make_sc_gather(d, n, _NUM_SEGMENTS, w)(x.T, ch)
    return out_t.T
